# baseline (speedup 1.0000x reference)
import functools

import jax
import jax.numpy as jnp
import numpy as np
from jax import lax
from jax.experimental import pallas as pl
from jax.experimental.pallas import tpu as pltpu
from jax.experimental.pallas import tpu_sc as plsc

SUB_KEYS = 512
TOP_K = 32
KEY_DIM = 256
VALUE_DIM = 128
INPUT_DIM = 1024
N_TOKENS = 2 * 4096

TB = 256
N_TILES = N_TOKENS // TB

NC = 2
NS = 16
NW = NC * NS
TOK_PER_W = N_TOKENS // NW
CHUNK_T = 4
CHUNK_R = CHUNK_T * TOP_K
N_CHUNKS = TOK_PER_W // CHUNK_T


_IDX_BITS = 0x1FF

_CAND_IJ = [(i, j) for i in range(TOP_K) for j in range(TOP_K // (i + 1))]
_N_CAND = len(_CAND_IJ)
_I_MAP = np.full((1, 128), -1, np.int32)
_J_MAP = np.full((1, 128), -1, np.int32)
for _c, (_i, _j) in enumerate(_CAND_IJ):
    _I_MAP[0, _c] = _i
    _J_MAP[0, _c] = _j


def _pack_keys(s, idx_bits):
    u = lax.bitcast_convert_type(s, jnp.int32)
    key = jnp.where(u >= 0, u, u ^ 0x7FFFFFFF)
    iota = lax.broadcasted_iota(jnp.int32, s.shape, 1)
    return (key & ~idx_bits) | iota


def _unpack_vals(kcat, idx_bits):
    keyv = kcat & ~idx_bits
    uv = jnp.where(keyv >= 0, keyv, keyv ^ 0x7FFFFFFF)
    return lax.bitcast_convert_type(uv, jnp.float32)


def _topk_packed_multi(arrs, k, idx_bits):
    keys = [_pack_keys(s, idx_bits) for s in arrs]
    minkey = jnp.int32(-(2**31))
    outs = [[] for _ in arrs]
    for _ in range(k):
        kmaxs = [jnp.max(kk, axis=1, keepdims=True) for kk in keys]
        keys = [jnp.where(kk == km, minkey, kk)
                for kk, km in zip(keys, kmaxs)]
        for o, km in zip(outs, kmaxs):
            o.append(km)
    kcats = [jnp.concatenate(o, axis=1) for o in outs]
    vals = [_unpack_vals(kc, idx_bits) for kc in kcats]
    idxs = [kc & idx_bits for kc in kcats]
    return vals, idxs


def _select_body(x_ref, wq_ref, bq_ref, ca_ref, cb_ref, imap_ref, jmap_ref,
                 fidx_ref, w_ref):
    x = x_ref[...]
    q = jnp.dot(x, wq_ref[...], preferred_element_type=jnp.float32) + bq_ref[...]
    qa = q[:, :KEY_DIM]
    qb = q[:, KEY_DIM:]
    sa = jnp.dot(qa, ca_ref[...], preferred_element_type=jnp.float32)
    sb = jnp.dot(qb, cb_ref[...], preferred_element_type=jnp.float32)
    half = TB // 2
    chains = [sa[:half], sa[half:], sb[:half], sb[half:]]
    vals, idxs = _topk_packed_multi(chains, TOP_K, _IDX_BITS)
    va = jnp.concatenate([vals[0], vals[1]], axis=0)
    vb = jnp.concatenate([vals[2], vals[3]], axis=0)
    ia = jnp.concatenate([idxs[0], idxs[1]], axis=0)
    ib = jnp.concatenate([idxs[2], idxs[3]], axis=0)
    row32 = lax.broadcasted_iota(jnp.int32, (TOP_K, 128), 0)
    ea = jnp.where(row32 == imap_ref[...], 1.0, 0.0).astype(jnp.float32)
    eb = jnp.where(row32 == jmap_ref[...], 1.0, 0.0).astype(jnp.float32)
    va_exp = jnp.dot(va, ea, preferred_element_type=jnp.float32)
    vb_exp = jnp.dot(vb, eb, preferred_element_type=jnp.float32)
    ia_exp = jnp.dot(ia.astype(jnp.float32), ea,
                     preferred_element_type=jnp.float32)
    ib_exp = jnp.dot(ib.astype(jnp.float32), eb,
                     preferred_element_type=jnp.float32)
    pos = lax.broadcasted_iota(jnp.int32, (TB, 128), 1)
    cs = jnp.where(pos < _N_CAND, va_exp + vb_exp, -jnp.inf)
    cidx_f = ia_exp * SUB_KEYS + ib_exp
    keys2 = _pack_keys(cs, 0x7F)
    minkey = jnp.int32(-(2**31))
    kmaxs2 = []
    for _ in range(TOP_K):
        km = jnp.max(keys2, axis=1, keepdims=True)
        keys2 = jnp.where(keys2 == km, minkey, keys2)
        kmaxs2.append(km)
    fv_cat = _unpack_vals(jnp.concatenate(kmaxs2, axis=1), 0x7F)
    fis = []
    for km in kmaxs2:
        eq = pos == (km & 0x7F)
        fis.append(jnp.sum(jnp.where(eq, cidx_f, 0.0), axis=1, keepdims=True))
    fidx_ref[...] = (jnp.concatenate(fis, axis=1) + 0.5).astype(jnp.int32)
    e = jnp.exp(fv_cat - fv_cat[:, 0:1])
    w_ref[...] = e / jnp.sum(e, axis=1, keepdims=True)


def _run_select(xf, wq, bq2, ca_t, cb_t, interpret=False):
    return pl.pallas_call(
        _select_body,
        grid=(xf.shape[0] // TB,),
        in_specs=[
            pl.BlockSpec((TB, INPUT_DIM), lambda i: (i, 0)),
            pl.BlockSpec((INPUT_DIM, 2 * KEY_DIM), lambda i: (0, 0)),
            pl.BlockSpec((1, 2 * KEY_DIM), lambda i: (0, 0)),
            pl.BlockSpec((KEY_DIM, SUB_KEYS), lambda i: (0, 0)),
            pl.BlockSpec((KEY_DIM, SUB_KEYS), lambda i: (0, 0)),
            pl.BlockSpec((1, 128), lambda i: (0, 0)),
            pl.BlockSpec((1, 128), lambda i: (0, 0)),
        ],
        out_specs=[
            pl.BlockSpec((TB, TOP_K), lambda i: (i, 0)),
            pl.BlockSpec((TB, TOP_K), lambda i: (i, 0)),
        ],
        out_shape=[
            jax.ShapeDtypeStruct((xf.shape[0], TOP_K), jnp.int32),
            jax.ShapeDtypeStruct((xf.shape[0], TOP_K), jnp.float32),
        ],
        interpret=interpret,
    )(xf, wq, bq2, ca_t, cb_t, jnp.asarray(_I_MAP), jnp.asarray(_J_MAP))


def _make_gather_body(tok_per_w, n_chunks):
  def _gather_body(idx_hbm, w_hbm, values_hbm, out_hbm, idx_v, w_v,
                   rows0_v, rows1_v, outc_v, sem0, sem1):
    wid = lax.axis_index("s") * NC + lax.axis_index("c")
    base_t = wid * tok_per_w
    pltpu.sync_copy(idx_hbm.at[wid], idx_v)
    pltpu.sync_copy(w_hbm.at[wid], w_v)

    def accumulate(c, rows_v, oslot):
        for tt in range(CHUNK_T):
            t_loc = c * CHUNK_T + tt
            w0 = w_v[t_loc, pl.ds(0, 16)]
            w1 = w_v[t_loc, pl.ds(16, 16)]
            accs = [jnp.zeros((16,), jnp.float32)
                    for _ in range(VALUE_DIM // 16)]
            for j in range(TOP_K):
                w = w0[j] if j < 16 else w1[j - 16]
                wv = jnp.full((16,), w, jnp.float32)
                r = tt * TOP_K + j
                for u in range(VALUE_DIM // 16):
                    accs[u] = accs[u] + wv * rows_v[r, pl.ds(u * 16, 16)]
            for u in range(VALUE_DIM // 16):
                outc_v[oslot * CHUNK_T + tt, pl.ds(u * 16, 16)] = accs[u]

    pltpu.async_copy(values_hbm.at[idx_v.at[0]], rows0_v, sem0)

    def pair_body(h, carry):
        c0 = 2 * h
        c1 = 2 * h + 1
        pltpu.async_copy(values_hbm.at[idx_v.at[c1]], rows1_v, sem1)
        pltpu.make_async_copy(values_hbm.at[idx_v.at[c0]], rows0_v, sem0).wait()
        accumulate(c0, rows0_v, 0)

        @pl.when(h + 1 < n_chunks // 2)
        def _():
            pltpu.async_copy(values_hbm.at[idx_v.at[c0 + 2]], rows0_v, sem0)

        pltpu.make_async_copy(values_hbm.at[idx_v.at[c1]], rows1_v, sem1).wait()
        accumulate(c1, rows1_v, 1)
        pltpu.sync_copy(
            outc_v, out_hbm.at[pl.ds(base_t + c0 * CHUNK_T, 2 * CHUNK_T)])
        return carry

    lax.fori_loop(0, n_chunks // 2, pair_body, 0)

  return _gather_body


def _gather_stage(fidx, w, values):
    n_tok = fidx.shape[0]
    tok_per_w = n_tok // NW
    n_chunks = tok_per_w // CHUNK_T
    idx_r = fidx.reshape(NW, n_chunks, CHUNK_R)
    w_r = w.reshape(NW, tok_per_w, TOP_K)
    mesh = plsc.VectorSubcoreMesh(core_axis_name="c", subcore_axis_name="s")
    run = functools.partial(
        pl.kernel,
        mesh=mesh,
        out_type=jax.ShapeDtypeStruct((n_tok, VALUE_DIM), jnp.float32),
        scratch_types=[
            pltpu.VMEM((n_chunks, CHUNK_R), jnp.int32),
            pltpu.VMEM((tok_per_w, TOP_K), jnp.float32),
            pltpu.VMEM((CHUNK_R, VALUE_DIM), jnp.float32),
            pltpu.VMEM((CHUNK_R, VALUE_DIM), jnp.float32),
            pltpu.VMEM((2 * CHUNK_T, VALUE_DIM), jnp.float32),
            pltpu.SemaphoreType.DMA,
            pltpu.SemaphoreType.DMA,
        ],
    )(_make_gather_body(tok_per_w, n_chunks))
    return run(idx_r, w_r, values)


def _out_body(x_ref, o_ref, wo_ref, bo_ref, g_ref, b_ref, y_ref):
    y = (x_ref[...]
         + jnp.dot(o_ref[...], wo_ref[...], preferred_element_type=jnp.float32)
         + bo_ref[...])
    mean = jnp.mean(y, axis=1, keepdims=True)
    yc = y - mean
    var = jnp.mean(yc * yc, axis=1, keepdims=True)
    yn = yc * lax.rsqrt(var + 1e-5)
    y_ref[...] = yn * g_ref[...] + b_ref[...]


def _run_out(xf, o, wo, bo2, g2, b2, interpret=False):
    return pl.pallas_call(
        _out_body,
        grid=(N_TILES,),
        in_specs=[
            pl.BlockSpec((TB, INPUT_DIM), lambda i: (i, 0)),
            pl.BlockSpec((TB, VALUE_DIM), lambda i: (i, 0)),
            pl.BlockSpec((VALUE_DIM, INPUT_DIM), lambda i: (0, 0)),
            pl.BlockSpec((1, INPUT_DIM), lambda i: (0, 0)),
            pl.BlockSpec((1, INPUT_DIM), lambda i: (0, 0)),
            pl.BlockSpec((1, INPUT_DIM), lambda i: (0, 0)),
        ],
        out_specs=pl.BlockSpec((TB, INPUT_DIM), lambda i: (i, 0)),
        out_shape=jax.ShapeDtypeStruct((N_TOKENS, INPUT_DIM), jnp.float32),
        interpret=interpret,
    )(xf, o, wo, bo2, g2, b2)


def kernel(x, W_q, b_q, codebook_a, codebook_b, values, W_out, b_out, ln_g, ln_b):
    batch, seq, _ = x.shape
    xf = x.reshape(N_TOKENS, INPUT_DIM)
    bq2 = b_q.reshape(1, -1)
    ca_t = codebook_a.T
    cb_t = codebook_b.T
    n_slices = 8
    sl = N_TOKENS // n_slices
    outs = []
    for s in range(n_slices):
        fidx_s, w_s = _run_select(xf[s * sl:(s + 1) * sl], W_q, bq2, ca_t, cb_t)
        outs.append(_gather_stage(fidx_s, w_s, values))
    o = jnp.concatenate(outs, axis=0)
    y = _run_out(xf, o, W_out, b_out.reshape(1, -1),
                 ln_g.reshape(1, -1), ln_b.reshape(1, -1))
    return y.reshape(batch, seq, INPUT_DIM)


# speedup vs baseline: 1.0058x; 1.0058x over previous
import functools

import jax
import jax.numpy as jnp
import numpy as np
from jax import lax
from jax.experimental import pallas as pl
from jax.experimental.pallas import tpu as pltpu
from jax.experimental.pallas import tpu_sc as plsc

SUB_KEYS = 512
TOP_K = 32
KEY_DIM = 256
VALUE_DIM = 128
INPUT_DIM = 1024
N_TOKENS = 2 * 4096

TB = 256
N_TILES = N_TOKENS // TB

NC = 2
NS = 16
NW = NC * NS
TOK_PER_W = N_TOKENS // NW
CHUNK_T = 4
CHUNK_R = CHUNK_T * TOP_K
N_CHUNKS = TOK_PER_W // CHUNK_T


_IDX_BITS = 0x1FF

_CAND_IJ = [(i, j) for i in range(TOP_K) for j in range(TOP_K // (i + 1))]
_N_CAND = len(_CAND_IJ)
_I_MAP = np.full((1, 128), -1, np.int32)
_J_MAP = np.full((1, 128), -1, np.int32)
for _c, (_i, _j) in enumerate(_CAND_IJ):
    _I_MAP[0, _c] = _i
    _J_MAP[0, _c] = _j


def _pack_keys(s, idx_bits):
    u = lax.bitcast_convert_type(s, jnp.int32)
    key = jnp.where(u >= 0, u, u ^ 0x7FFFFFFF)
    iota = lax.broadcasted_iota(jnp.int32, s.shape, 1)
    return (key & ~idx_bits) | iota


def _unpack_vals(kcat, idx_bits):
    keyv = kcat & ~idx_bits
    uv = jnp.where(keyv >= 0, keyv, keyv ^ 0x7FFFFFFF)
    return lax.bitcast_convert_type(uv, jnp.float32)


def _topk_packed_multi(arrs, k, idx_bits):
    keys = [_pack_keys(s, idx_bits) for s in arrs]
    minkey = jnp.int32(-(2**31))
    outs = [[] for _ in arrs]
    for _ in range(k):
        kmaxs = [jnp.max(kk, axis=1, keepdims=True) for kk in keys]
        keys = [jnp.where(kk == km, minkey, kk)
                for kk, km in zip(keys, kmaxs)]
        for o, km in zip(outs, kmaxs):
            o.append(km)
    kcats = [jnp.concatenate(o, axis=1) for o in outs]
    vals = [_unpack_vals(kc, idx_bits) for kc in kcats]
    idxs = [kc & idx_bits for kc in kcats]
    return vals, idxs


def _select_body(x_ref, wq_ref, bq_ref, ca_ref, cb_ref, imap_ref, jmap_ref,
                 fidx_ref, w_ref):
    x = x_ref[...]
    q = jnp.dot(x, wq_ref[...], preferred_element_type=jnp.float32) + bq_ref[...]
    qa = q[:, :KEY_DIM]
    qb = q[:, KEY_DIM:]
    sa = jnp.dot(qa, ca_ref[...], preferred_element_type=jnp.float32)
    sb = jnp.dot(qb, cb_ref[...], preferred_element_type=jnp.float32)
    half = TB // 2
    chains = [sa[:half], sa[half:], sb[:half], sb[half:]]
    vals, idxs = _topk_packed_multi(chains, TOP_K, _IDX_BITS)
    va = jnp.concatenate([vals[0], vals[1]], axis=0)
    vb = jnp.concatenate([vals[2], vals[3]], axis=0)
    ia = jnp.concatenate([idxs[0], idxs[1]], axis=0)
    ib = jnp.concatenate([idxs[2], idxs[3]], axis=0)
    row32 = lax.broadcasted_iota(jnp.int32, (TOP_K, 128), 0)
    ea = jnp.where(row32 == imap_ref[...], 1.0, 0.0).astype(jnp.float32)
    eb = jnp.where(row32 == jmap_ref[...], 1.0, 0.0).astype(jnp.float32)
    va_exp = jnp.dot(va, ea, preferred_element_type=jnp.float32)
    vb_exp = jnp.dot(vb, eb, preferred_element_type=jnp.float32)
    ia_exp = jnp.dot(ia.astype(jnp.float32), ea,
                     preferred_element_type=jnp.float32)
    ib_exp = jnp.dot(ib.astype(jnp.float32), eb,
                     preferred_element_type=jnp.float32)
    pos = lax.broadcasted_iota(jnp.int32, (TB, 128), 1)
    cs = jnp.where(pos < _N_CAND, va_exp + vb_exp, -jnp.inf)
    cidx_f = ia_exp * SUB_KEYS + ib_exp
    keys2 = _pack_keys(cs, 0x7F)
    minkey = jnp.int32(-(2**31))
    kmaxs2 = []
    for _ in range(TOP_K):
        km = jnp.max(keys2, axis=1, keepdims=True)
        keys2 = jnp.where(keys2 == km, minkey, keys2)
        kmaxs2.append(km)
    fv_cat = _unpack_vals(jnp.concatenate(kmaxs2, axis=1), 0x7F)
    fis = []
    for km in kmaxs2:
        eq = pos == (km & 0x7F)
        fis.append(jnp.sum(jnp.where(eq, cidx_f, 0.0), axis=1, keepdims=True))
    fidx_ref[...] = (jnp.concatenate(fis, axis=1) + 0.5).astype(jnp.int32)
    e = jnp.exp(fv_cat - fv_cat[:, 0:1])
    w_ref[...] = e / jnp.sum(e, axis=1, keepdims=True)


def _run_select(xf, wq, bq2, ca_t, cb_t, interpret=False):
    return pl.pallas_call(
        _select_body,
        grid=(xf.shape[0] // TB,),
        in_specs=[
            pl.BlockSpec((TB, INPUT_DIM), lambda i: (i, 0)),
            pl.BlockSpec((INPUT_DIM, 2 * KEY_DIM), lambda i: (0, 0)),
            pl.BlockSpec((1, 2 * KEY_DIM), lambda i: (0, 0)),
            pl.BlockSpec((KEY_DIM, SUB_KEYS), lambda i: (0, 0)),
            pl.BlockSpec((KEY_DIM, SUB_KEYS), lambda i: (0, 0)),
            pl.BlockSpec((1, 128), lambda i: (0, 0)),
            pl.BlockSpec((1, 128), lambda i: (0, 0)),
        ],
        out_specs=[
            pl.BlockSpec((TB, TOP_K), lambda i: (i, 0)),
            pl.BlockSpec((TB, TOP_K), lambda i: (i, 0)),
        ],
        out_shape=[
            jax.ShapeDtypeStruct((xf.shape[0], TOP_K), jnp.int32),
            jax.ShapeDtypeStruct((xf.shape[0], TOP_K), jnp.float32),
        ],
        interpret=interpret,
    )(xf, wq, bq2, ca_t, cb_t, jnp.asarray(_I_MAP), jnp.asarray(_J_MAP))


def _make_gather_body(tok_per_w, n_chunks):
  def _gather_body(idx_hbm, w_hbm, values_hbm, out_hbm, idx_v, w_v,
                   rows0_v, rows1_v, outc_v, sem0, sem1):
    wid = lax.axis_index("s") * NC + lax.axis_index("c")
    base_t = wid * tok_per_w
    pltpu.sync_copy(idx_hbm.at[wid], idx_v)
    pltpu.sync_copy(w_hbm.at[wid], w_v)

    def accumulate(c, rows_v, oslot):
        for tt in range(CHUNK_T):
            t_loc = c * CHUNK_T + tt
            w0 = w_v[t_loc, pl.ds(0, 16)]
            w1 = w_v[t_loc, pl.ds(16, 16)]
            accs = [jnp.zeros((16,), jnp.float32)
                    for _ in range(VALUE_DIM // 16)]
            for j in range(TOP_K):
                w = w0[j] if j < 16 else w1[j - 16]
                wv = jnp.full((16,), w, jnp.float32)
                r = tt * TOP_K + j
                for u in range(VALUE_DIM // 16):
                    accs[u] = accs[u] + wv * rows_v[r, pl.ds(u * 16, 16)]
            for u in range(VALUE_DIM // 16):
                outc_v[oslot * CHUNK_T + tt, pl.ds(u * 16, 16)] = accs[u]

    pltpu.async_copy(values_hbm.at[idx_v.at[0]], rows0_v, sem0)

    def pair_body(h, carry):
        c0 = 2 * h
        c1 = 2 * h + 1
        pltpu.async_copy(values_hbm.at[idx_v.at[c1]], rows1_v, sem1)
        pltpu.make_async_copy(values_hbm.at[idx_v.at[c0]], rows0_v, sem0).wait()
        accumulate(c0, rows0_v, 0)

        @pl.when(h + 1 < n_chunks // 2)
        def _():
            pltpu.async_copy(values_hbm.at[idx_v.at[c0 + 2]], rows0_v, sem0)

        pltpu.make_async_copy(values_hbm.at[idx_v.at[c1]], rows1_v, sem1).wait()
        accumulate(c1, rows1_v, 1)
        pltpu.sync_copy(
            outc_v, out_hbm.at[pl.ds(base_t + c0 * CHUNK_T, 2 * CHUNK_T)])
        return carry

    lax.fori_loop(0, n_chunks // 2, pair_body, 0)

  return _gather_body


def _gather_stage(fidx, w, values):
    n_tok = fidx.shape[0]
    tok_per_w = n_tok // NW
    n_chunks = tok_per_w // CHUNK_T
    idx_r = fidx.reshape(NW, n_chunks, CHUNK_R)
    w_r = w.reshape(NW, tok_per_w, TOP_K)
    mesh = plsc.VectorSubcoreMesh(core_axis_name="c", subcore_axis_name="s")
    run = functools.partial(
        pl.kernel,
        mesh=mesh,
        out_type=jax.ShapeDtypeStruct((n_tok, VALUE_DIM), jnp.float32),
        scratch_types=[
            pltpu.VMEM((n_chunks, CHUNK_R), jnp.int32),
            pltpu.VMEM((tok_per_w, TOP_K), jnp.float32),
            pltpu.VMEM((CHUNK_R, VALUE_DIM), jnp.float32),
            pltpu.VMEM((CHUNK_R, VALUE_DIM), jnp.float32),
            pltpu.VMEM((2 * CHUNK_T, VALUE_DIM), jnp.float32),
            pltpu.SemaphoreType.DMA,
            pltpu.SemaphoreType.DMA,
        ],
    )(_make_gather_body(tok_per_w, n_chunks))
    return run(idx_r, w_r, values)


def _out_body(x_ref, o_ref, wo_ref, bo_ref, g_ref, b_ref, y_ref):
    y = (x_ref[...]
         + jnp.dot(o_ref[...], wo_ref[...], preferred_element_type=jnp.float32)
         + bo_ref[...])
    mean = jnp.mean(y, axis=1, keepdims=True)
    yc = y - mean
    var = jnp.mean(yc * yc, axis=1, keepdims=True)
    yn = yc * lax.rsqrt(var + 1e-5)
    y_ref[...] = yn * g_ref[...] + b_ref[...]


def _run_out(xf, o, wo, bo2, g2, b2, interpret=False):
    return pl.pallas_call(
        _out_body,
        grid=(N_TILES,),
        in_specs=[
            pl.BlockSpec((TB, INPUT_DIM), lambda i: (i, 0)),
            pl.BlockSpec((TB, VALUE_DIM), lambda i: (i, 0)),
            pl.BlockSpec((VALUE_DIM, INPUT_DIM), lambda i: (0, 0)),
            pl.BlockSpec((1, INPUT_DIM), lambda i: (0, 0)),
            pl.BlockSpec((1, INPUT_DIM), lambda i: (0, 0)),
            pl.BlockSpec((1, INPUT_DIM), lambda i: (0, 0)),
        ],
        out_specs=pl.BlockSpec((TB, INPUT_DIM), lambda i: (i, 0)),
        out_shape=jax.ShapeDtypeStruct((N_TOKENS, INPUT_DIM), jnp.float32),
        interpret=interpret,
    )(xf, o, wo, bo2, g2, b2)


def kernel(x, W_q, b_q, codebook_a, codebook_b, values, W_out, b_out, ln_g, ln_b):
    batch, seq, _ = x.shape
    xf = x.reshape(N_TOKENS, INPUT_DIM)
    bq2 = b_q.reshape(1, -1)
    ca_t = codebook_a.T
    cb_t = codebook_b.T
    n_slices = 4
    sl = N_TOKENS // n_slices
    outs = []
    for s in range(n_slices):
        fidx_s, w_s = _run_select(xf[s * sl:(s + 1) * sl], W_q, bq2, ca_t, cb_t)
        outs.append(_gather_stage(fidx_s, w_s, values))
    o = jnp.concatenate(outs, axis=0)
    y = _run_out(xf, o, W_out, b_out.reshape(1, -1),
                 ln_g.reshape(1, -1), ln_b.reshape(1, -1))
    return y.reshape(batch, seq, INPUT_DIM)


# speedup vs baseline: 1.0628x; 1.0566x over previous
import functools

import jax
import jax.numpy as jnp
import numpy as np
from jax import lax
from jax.experimental import pallas as pl
from jax.experimental.pallas import tpu as pltpu
from jax.experimental.pallas import tpu_sc as plsc

SUB_KEYS = 512
TOP_K = 32
KEY_DIM = 256
VALUE_DIM = 128
INPUT_DIM = 1024
N_TOKENS = 2 * 4096

TB = 512
N_TILES = N_TOKENS // TB

NC = 2
NS = 16
NW = NC * NS
TOK_PER_W = N_TOKENS // NW
CHUNK_T = 4
CHUNK_R = CHUNK_T * TOP_K
N_CHUNKS = TOK_PER_W // CHUNK_T


_IDX_BITS = 0x1FF

_CAND_IJ = [(i, j) for i in range(TOP_K) for j in range(TOP_K // (i + 1))]
_N_CAND = len(_CAND_IJ)
_I_MAP = np.full((1, 128), -1, np.int32)
_J_MAP = np.full((1, 128), -1, np.int32)
for _c, (_i, _j) in enumerate(_CAND_IJ):
    _I_MAP[0, _c] = _i
    _J_MAP[0, _c] = _j


def _pack_keys(s, idx_bits):
    u = lax.bitcast_convert_type(s, jnp.int32)
    key = jnp.where(u >= 0, u, u ^ 0x7FFFFFFF)
    iota = lax.broadcasted_iota(jnp.int32, s.shape, 1)
    return (key & ~idx_bits) | iota


def _unpack_vals(kcat, idx_bits):
    keyv = kcat & ~idx_bits
    uv = jnp.where(keyv >= 0, keyv, keyv ^ 0x7FFFFFFF)
    return lax.bitcast_convert_type(uv, jnp.float32)


def _topk_packed_multi(arrs, k, idx_bits):
    keys = [_pack_keys(s, idx_bits) for s in arrs]
    minkey = jnp.int32(-(2**31))
    outs = [[] for _ in arrs]
    for _ in range(k):
        kmaxs = [jnp.max(kk, axis=1, keepdims=True) for kk in keys]
        keys = [jnp.where(kk == km, minkey, kk)
                for kk, km in zip(keys, kmaxs)]
        for o, km in zip(outs, kmaxs):
            o.append(km)
    kcats = [jnp.concatenate(o, axis=1) for o in outs]
    vals = [_unpack_vals(kc, idx_bits) for kc in kcats]
    idxs = [kc & idx_bits for kc in kcats]
    return vals, idxs


def _select_body(x_ref, wq_ref, bq_ref, ca_ref, cb_ref, imap_ref, jmap_ref,
                 fidx_ref, w_ref):
    x = x_ref[...]
    q = jnp.dot(x, wq_ref[...], preferred_element_type=jnp.float32) + bq_ref[...]
    qa = q[:, :KEY_DIM]
    qb = q[:, KEY_DIM:]
    sa = jnp.dot(qa, ca_ref[...], preferred_element_type=jnp.float32)
    sb = jnp.dot(qb, cb_ref[...], preferred_element_type=jnp.float32)
    half = TB // 2
    chains = [sa[:half], sa[half:], sb[:half], sb[half:]]
    vals, idxs = _topk_packed_multi(chains, TOP_K, _IDX_BITS)
    va = jnp.concatenate([vals[0], vals[1]], axis=0)
    vb = jnp.concatenate([vals[2], vals[3]], axis=0)
    ia = jnp.concatenate([idxs[0], idxs[1]], axis=0)
    ib = jnp.concatenate([idxs[2], idxs[3]], axis=0)
    row32 = lax.broadcasted_iota(jnp.int32, (TOP_K, 128), 0)
    ea = jnp.where(row32 == imap_ref[...], 1.0, 0.0).astype(jnp.float32)
    eb = jnp.where(row32 == jmap_ref[...], 1.0, 0.0).astype(jnp.float32)
    va_exp = jnp.dot(va, ea, preferred_element_type=jnp.float32)
    vb_exp = jnp.dot(vb, eb, preferred_element_type=jnp.float32)
    ia_exp = jnp.dot(ia.astype(jnp.float32), ea,
                     preferred_element_type=jnp.float32)
    ib_exp = jnp.dot(ib.astype(jnp.float32), eb,
                     preferred_element_type=jnp.float32)
    pos = lax.broadcasted_iota(jnp.int32, (TB, 128), 1)
    cs = jnp.where(pos < _N_CAND, va_exp + vb_exp, -jnp.inf)
    cidx_f = ia_exp * SUB_KEYS + ib_exp
    keys2 = _pack_keys(cs, 0x7F)
    minkey = jnp.int32(-(2**31))
    kmaxs2 = []
    for _ in range(TOP_K):
        km = jnp.max(keys2, axis=1, keepdims=True)
        keys2 = jnp.where(keys2 == km, minkey, keys2)
        kmaxs2.append(km)
    fv_cat = _unpack_vals(jnp.concatenate(kmaxs2, axis=1), 0x7F)
    fis = []
    for km in kmaxs2:
        eq = pos == (km & 0x7F)
        fis.append(jnp.sum(jnp.where(eq, cidx_f, 0.0), axis=1, keepdims=True))
    fidx_ref[...] = (jnp.concatenate(fis, axis=1) + 0.5).astype(jnp.int32)
    e = jnp.exp(fv_cat - fv_cat[:, 0:1])
    w_ref[...] = e / jnp.sum(e, axis=1, keepdims=True)


def _run_select(xf, wq, bq2, ca_t, cb_t, interpret=False):
    return pl.pallas_call(
        _select_body,
        grid=(xf.shape[0] // TB,),
        in_specs=[
            pl.BlockSpec((TB, INPUT_DIM), lambda i: (i, 0)),
            pl.BlockSpec((INPUT_DIM, 2 * KEY_DIM), lambda i: (0, 0)),
            pl.BlockSpec((1, 2 * KEY_DIM), lambda i: (0, 0)),
            pl.BlockSpec((KEY_DIM, SUB_KEYS), lambda i: (0, 0)),
            pl.BlockSpec((KEY_DIM, SUB_KEYS), lambda i: (0, 0)),
            pl.BlockSpec((1, 128), lambda i: (0, 0)),
            pl.BlockSpec((1, 128), lambda i: (0, 0)),
        ],
        out_specs=[
            pl.BlockSpec((TB, TOP_K), lambda i: (i, 0)),
            pl.BlockSpec((TB, TOP_K), lambda i: (i, 0)),
        ],
        out_shape=[
            jax.ShapeDtypeStruct((xf.shape[0], TOP_K), jnp.int32),
            jax.ShapeDtypeStruct((xf.shape[0], TOP_K), jnp.float32),
        ],
        interpret=interpret,
    )(xf, wq, bq2, ca_t, cb_t, jnp.asarray(_I_MAP), jnp.asarray(_J_MAP))


def _make_gather_body(tok_per_w, n_chunks):
  def _gather_body(idx_hbm, w_hbm, values_hbm, out_hbm, idx_v, w_v,
                   rows0_v, rows1_v, outc_v, sem0, sem1):
    wid = lax.axis_index("s") * NC + lax.axis_index("c")
    base_t = wid * tok_per_w
    pltpu.sync_copy(idx_hbm.at[wid], idx_v)
    pltpu.sync_copy(w_hbm.at[wid], w_v)

    def accumulate(c, rows_v, oslot):
        for tt in range(CHUNK_T):
            t_loc = c * CHUNK_T + tt
            w0 = w_v[t_loc, pl.ds(0, 16)]
            w1 = w_v[t_loc, pl.ds(16, 16)]
            accs = [jnp.zeros((16,), jnp.float32)
                    for _ in range(VALUE_DIM // 16)]
            for j in range(TOP_K):
                w = w0[j] if j < 16 else w1[j - 16]
                wv = jnp.full((16,), w, jnp.float32)
                r = tt * TOP_K + j
                for u in range(VALUE_DIM // 16):
                    accs[u] = accs[u] + wv * rows_v[r, pl.ds(u * 16, 16)]
            for u in range(VALUE_DIM // 16):
                outc_v[oslot * CHUNK_T + tt, pl.ds(u * 16, 16)] = accs[u]

    pltpu.async_copy(values_hbm.at[idx_v.at[0]], rows0_v, sem0)

    def pair_body(h, carry):
        c0 = 2 * h
        c1 = 2 * h + 1
        pltpu.async_copy(values_hbm.at[idx_v.at[c1]], rows1_v, sem1)
        pltpu.make_async_copy(values_hbm.at[idx_v.at[c0]], rows0_v, sem0).wait()
        accumulate(c0, rows0_v, 0)

        @pl.when(h + 1 < n_chunks // 2)
        def _():
            pltpu.async_copy(values_hbm.at[idx_v.at[c0 + 2]], rows0_v, sem0)

        pltpu.make_async_copy(values_hbm.at[idx_v.at[c1]], rows1_v, sem1).wait()
        accumulate(c1, rows1_v, 1)
        pltpu.sync_copy(
            outc_v, out_hbm.at[pl.ds(base_t + c0 * CHUNK_T, 2 * CHUNK_T)])
        return carry

    lax.fori_loop(0, n_chunks // 2, pair_body, 0)

  return _gather_body


def _gather_stage(fidx, w, values):
    n_tok = fidx.shape[0]
    tok_per_w = n_tok // NW
    n_chunks = tok_per_w // CHUNK_T
    idx_r = fidx.reshape(NW, n_chunks, CHUNK_R)
    w_r = w.reshape(NW, tok_per_w, TOP_K)
    mesh = plsc.VectorSubcoreMesh(core_axis_name="c", subcore_axis_name="s")
    run = functools.partial(
        pl.kernel,
        mesh=mesh,
        out_type=jax.ShapeDtypeStruct((n_tok, VALUE_DIM), jnp.float32),
        scratch_types=[
            pltpu.VMEM((n_chunks, CHUNK_R), jnp.int32),
            pltpu.VMEM((tok_per_w, TOP_K), jnp.float32),
            pltpu.VMEM((CHUNK_R, VALUE_DIM), jnp.float32),
            pltpu.VMEM((CHUNK_R, VALUE_DIM), jnp.float32),
            pltpu.VMEM((2 * CHUNK_T, VALUE_DIM), jnp.float32),
            pltpu.SemaphoreType.DMA,
            pltpu.SemaphoreType.DMA,
        ],
    )(_make_gather_body(tok_per_w, n_chunks))
    return run(idx_r, w_r, values)


def _out_body(x_ref, o_ref, wo_ref, bo_ref, g_ref, b_ref, y_ref):
    y = (x_ref[...]
         + jnp.dot(o_ref[...], wo_ref[...], preferred_element_type=jnp.float32)
         + bo_ref[...])
    mean = jnp.mean(y, axis=1, keepdims=True)
    yc = y - mean
    var = jnp.mean(yc * yc, axis=1, keepdims=True)
    yn = yc * lax.rsqrt(var + 1e-5)
    y_ref[...] = yn * g_ref[...] + b_ref[...]


def _run_out(xf, o, wo, bo2, g2, b2, interpret=False):
    return pl.pallas_call(
        _out_body,
        grid=(N_TILES,),
        in_specs=[
            pl.BlockSpec((TB, INPUT_DIM), lambda i: (i, 0)),
            pl.BlockSpec((TB, VALUE_DIM), lambda i: (i, 0)),
            pl.BlockSpec((VALUE_DIM, INPUT_DIM), lambda i: (0, 0)),
            pl.BlockSpec((1, INPUT_DIM), lambda i: (0, 0)),
            pl.BlockSpec((1, INPUT_DIM), lambda i: (0, 0)),
            pl.BlockSpec((1, INPUT_DIM), lambda i: (0, 0)),
        ],
        out_specs=pl.BlockSpec((TB, INPUT_DIM), lambda i: (i, 0)),
        out_shape=jax.ShapeDtypeStruct((N_TOKENS, INPUT_DIM), jnp.float32),
        interpret=interpret,
    )(xf, o, wo, bo2, g2, b2)


def kernel(x, W_q, b_q, codebook_a, codebook_b, values, W_out, b_out, ln_g, ln_b):
    batch, seq, _ = x.shape
    xf = x.reshape(N_TOKENS, INPUT_DIM)
    bq2 = b_q.reshape(1, -1)
    ca_t = codebook_a.T
    cb_t = codebook_b.T
    n_slices = 4
    sl = N_TOKENS // n_slices
    outs = []
    for s in range(n_slices):
        fidx_s, w_s = _run_select(xf[s * sl:(s + 1) * sl], W_q, bq2, ca_t, cb_t)
        outs.append(_gather_stage(fidx_s, w_s, values))
    o = jnp.concatenate(outs, axis=0)
    y = _run_out(xf, o, W_out, b_out.reshape(1, -1),
                 ln_g.reshape(1, -1), ln_b.reshape(1, -1))
    return y.reshape(batch, seq, INPUT_DIM)
